# baseline (speedup 1.0000x reference)
import functools

import jax
import jax.numpy as jnp
from jax import lax
from jax.experimental import pallas as pl
from jax.experimental.pallas import tpu as pltpu
from jax.experimental.pallas import tpu_sc as plsc

N_GRAPHS = 16
_F32 = jnp.float32

N_PAD = 10240
SC_CORES = 2
SC_TILES = 16
N_WORKERS = SC_CORES * SC_TILES
CHUNK = 128
NBUF = 16
PF = 8
ROWS_PER_TILE = N_PAD // SC_TILES


def _sc_mesh():
    return plsc.VectorSubcoreMesh(core_axis_name="c", subcore_axis_name="s")


def _spmm_sc(n_chunks):
    ept_rows = n_chunks

    def body(v_hbm, src_hbm, dst_hbm, zz_hbm, out_hbm, acc, sidx, didx,
             gb, gsem):
        c = lax.axis_index("c")
        s = lax.axis_index("s")
        wid = c * SC_TILES + s
        pltpu.sync_copy(zz_hbm.at[pl.ds(s * ROWS_PER_TILE, ROWS_PER_TILE)],
                        acc.at[pl.ds(s * ROWS_PER_TILE, ROWS_PER_TILE)])
        pltpu.sync_copy(src_hbm.at[pl.ds(wid * ept_rows, ept_rows)], sidx)
        pltpu.sync_copy(dst_hbm.at[pl.ds(wid * ept_rows, ept_rows)], didx)
        plsc.subcore_barrier()

        for b in range(PF):
            pltpu.make_async_copy(v_hbm.at[sidx.at[b]], gb.at[b],
                                  gsem.at[b]).start()

        def step(j, carry):
            for b in range(NBUF):
                jj = j + b
                bp = (b + PF) % NBUF
                pltpu.make_async_copy(v_hbm.at[sidx.at[jj]], gb.at[b],
                                      gsem.at[b]).wait()

                @pl.when(jj + PF < n_chunks)
                def _():
                    pltpu.make_async_copy(v_hbm.at[sidx.at[jj + PF]],
                                          gb.at[bp], gsem.at[bp]).start()

                pltpu.sync_copy(gb.at[b], acc.at[didx.at[jj]], add=True)
            return carry

        lax.fori_loop(0, n_chunks // NBUF, lambda i, cc: step(i * NBUF, cc),
                      0, unroll=False)
        plsc.subcore_barrier()
        pltpu.sync_copy(acc.at[pl.ds(s * ROWS_PER_TILE, ROWS_PER_TILE)],
                        out_hbm.at[c].at[pl.ds(s * ROWS_PER_TILE,
                                               ROWS_PER_TILE)])

    return pl.kernel(
        body,
        out_type=jax.ShapeDtypeStruct((SC_CORES, N_PAD, 32), _F32),
        mesh=_sc_mesh(),
        compiler_params=pltpu.CompilerParams(use_tc_tiling_on_sc=False),
        scratch_types=[
            pltpu.VMEM_SHARED((N_PAD, 32), _F32),
            pltpu.VMEM((ept_rows, CHUNK), jnp.int32),
            pltpu.VMEM((ept_rows, CHUNK), jnp.int32),
            pltpu.VMEM((NBUF, CHUNK, 32), _F32),
            pltpu.SemaphoreType.DMA((NBUF,)),
        ],
    )


def _deg_sc(n_chunks):
    ept_rows = n_chunks

    def body(dst_hbm, zz_hbm, out_hbm, acc, didx, ones):
        c = lax.axis_index("c")
        s = lax.axis_index("s")
        wid = c * SC_TILES + s
        pltpu.sync_copy(zz_hbm.at[pl.ds(s * ROWS_PER_TILE, ROWS_PER_TILE)],
                        acc.at[pl.ds(s * ROWS_PER_TILE, ROWS_PER_TILE)])
        pltpu.sync_copy(dst_hbm.at[pl.ds(wid * ept_rows, ept_rows)], didx)

        def initones(i, cc):
            ones[i, :] = jnp.ones((16,), _F32)
            return cc

        lax.fori_loop(0, CHUNK, initones, 0, unroll=False)
        plsc.subcore_barrier()

        def step(j, cc):
            pltpu.sync_copy(ones, acc.at[didx.at[j]], add=True)
            return cc

        lax.fori_loop(0, n_chunks, step, 0, unroll=False)
        plsc.subcore_barrier()
        pltpu.sync_copy(acc.at[pl.ds(s * ROWS_PER_TILE, ROWS_PER_TILE)],
                        out_hbm.at[c].at[pl.ds(s * ROWS_PER_TILE,
                                               ROWS_PER_TILE)])

    return pl.kernel(
        body,
        out_type=jax.ShapeDtypeStruct((SC_CORES, N_PAD, 16), _F32),
        mesh=_sc_mesh(),
        compiler_params=pltpu.CompilerParams(use_tc_tiling_on_sc=False),
        scratch_types=[
            pltpu.VMEM_SHARED((N_PAD, 16), _F32),
            pltpu.VMEM((ept_rows, CHUNK), jnp.int32),
            pltpu.VMEM((CHUNK, 16), _F32),
        ],
    )


def _batch_onehot(batch2d):
    gid = lax.broadcasted_iota(jnp.int32, (1, N_GRAPHS), 1)
    bmat = (batch2d == gid).astype(_F32)
    cnt = jnp.maximum(jnp.sum(bmat, axis=0, keepdims=True), 1.0)
    inv_cnt = (1.0 / cnt).reshape(N_GRAPHS, 1)
    return bmat, inv_cnt


def _dotg(a, b, ca, cb):
    return lax.dot_general(a, b, (((ca,), (cb,)), ((), ())),
                           preferred_element_type=_F32)


def _split(a):
    hi = a.astype(jnp.bfloat16).astype(_F32)
    return hi, a - hi


def _dot_oh(oh, b, ca, cb):
    bh, bl = _split(b)
    return _dotg(oh, bh, ca, cb) + _dotg(oh, bl, ca, cb)


def _dot3(a, b, ca, cb):
    ah, al = _split(a)
    bh, bl = _split(b)
    return (_dotg(ah, bh, ca, cb) + _dotg(ah, bl, ca, cb)
            + _dotg(al, bh, ca, cb))


def _gnorm(z, bmat, inv_cnt, w, b, ms):
    mean = _dot_oh(bmat, z, 0, 0) * inv_cnt
    o = z - _dot_oh(bmat, mean, 1, 0) * ms
    var = _dot_oh(bmat, o * o, 0, 0) * inv_cnt
    rstd = lax.rsqrt(var + 1e-5)
    return w * o * _dot_oh(bmat, rstd, 1, 0) + b


def _pre_body(deg_ref, x_ref, w0_ref, dis_ref, v0_ref):
    dis = lax.rsqrt(deg_ref[...] + 1.0)
    dis_ref[...] = dis
    v0_ref[...] = dis * _dot3(x_ref[...], w0_ref[...], 1, 0)


def _d0_body(s_ref, v_ref, dis_ref, batch_ref, b0_ref, gw_ref, gb_ref,
             gms_ref, h_ref, vh_ref):
    bmat, inv_cnt = _batch_onehot(batch_ref[...])
    a = dis_ref[...] * (s_ref[...] + v_ref[...]) + b0_ref[...]
    h = jax.nn.relu(_gnorm(a, bmat, inv_cnt, gw_ref[...], gb_ref[...],
                           gms_ref[...]))
    h_ref[...] = h
    vh_ref[...] = dis_ref[...] * h


def _d1_body(s_ref, vh_ref, dis_ref, batch_ref, w1_ref, b1_ref, gw_ref,
             gb_ref, gms_ref, w2_ref, v2_ref):
    bmat, inv_cnt = _batch_onehot(batch_ref[...])
    a = dis_ref[...] * (s_ref[...] + vh_ref[...])
    z = _dot3(a, w1_ref[...], 1, 0) + b1_ref[...]
    t = jax.nn.relu(_gnorm(z, bmat, inv_cnt, gw_ref[...], gb_ref[...],
                           gms_ref[...]))
    v2_ref[...] = dis_ref[...] * _dot3(t, w2_ref[...], 1, 0)


def _d2_body(relu_feat, s_ref, v2_ref, dis_ref, batch_ref, b2_ref, gw_ref,
             gb_ref, gms_ref, feat_ref, feato_ref, vh_ref):
    bmat, inv_cnt = _batch_onehot(batch_ref[...])
    a = dis_ref[...] * (s_ref[...] + v2_ref[...]) + b2_ref[...]
    t2 = jax.nn.relu(_gnorm(a, bmat, inv_cnt, gw_ref[...], gb_ref[...],
                            gms_ref[...]))
    fin = jax.nn.relu(feat_ref[...]) if relu_feat else feat_ref[...]
    f = fin + t2
    feato_ref[...] = f
    vh_ref[...] = dis_ref[...] * jax.nn.relu(f)


def _final_body(feat_ref, batch_ref, wvt_ref, bv_ref, owt_ref, ob_ref,
                out_ref):
    h = jax.nn.relu(feat_ref[...])
    batch2d = batch_ref[...]
    bmat, _ = _batch_onehot(batch2d)
    rows = []
    for g in range(N_GRAPHS):
        hg = jnp.where(batch2d == g, h, -1.0)
        rows.append(jnp.max(hg, axis=0, keepdims=True))
    pooled = jnp.concatenate(rows, axis=0)
    go = _dot3(_dot3(pooled, wvt_ref[...], 1, 0) + bv_ref[...],
               owt_ref[...], 1, 0) + ob_ref[...]
    out_ref[...] = _dot_oh(bmat, go, 1, 0)


def _call(body, out_shapes, *args):
    return pl.pallas_call(
        body,
        out_shape=[jax.ShapeDtypeStruct(s, _F32) for s in out_shapes],
    )(*args)


def kernel(x, edge_index, batch, params):
    N = x.shape[0]
    E = edge_index.shape[1]
    NP = N_PAD

    xp = jnp.zeros((NP, x.shape[1]), _F32).at[:N].set(x)
    batch2d = jnp.concatenate(
        [batch, jnp.full((NP - N,), N_GRAPHS, batch.dtype)])[:, None]
    n_chunks = -(-E // (CHUNK * N_WORKERS))
    n_chunks = -(-n_chunks // NBUF) * NBUF
    e_pad = n_chunks * CHUNK * N_WORKERS
    idx_pad = jnp.full((2, e_pad - E), N, edge_index.dtype)
    srcp = jnp.concatenate([edge_index, idx_pad], axis=1)[0].reshape(
        N_WORKERS * n_chunks, CHUNK)
    dstp = jnp.concatenate([edge_index, idx_pad], axis=1)[1].reshape(
        N_WORKERS * n_chunks, CHUNK)
    zz32 = jnp.zeros((NP, 32), _F32)
    zz16 = jnp.zeros((NP, 16), _F32)

    spmm = _spmm_sc(n_chunks)
    degk = _deg_sc(n_chunks)

    def scat(v):
        sp = spmm(v, srcp, dstp, zz32)
        return sp[0] + sp[1]

    deg_part = degk(dstp, zz16)
    degsum = (deg_part[0, :, 0] + deg_part[1, :, 0])[:, None]

    p = params
    dis, v0 = _call(_pre_body, [(NP, 1), (NP, 32)], degsum, xp,
                    p["conv0_W"])

    s0 = scat(v0)
    h, vh = _call(
        _d0_body, [(NP, 32), (NP, 32)], s0, v0, dis, batch2d,
        p["conv0_b"][None, :], p["gn0_w"][None, :], p["gn0_b"][None, :],
        p["gn0_ms"][None, :])

    feat = h
    for bi, block in enumerate(p["blocks"]):
        for li, lp in enumerate(block):
            s1 = scat(vh)
            (v2,) = _call(
                _d1_body, [(NP, 32)], s1, vh, dis, batch2d, lp["W1"],
                lp["b1"][None, :], lp["gn1_w"][None, :],
                lp["gn1_b"][None, :], lp["gn1_ms"][None, :], lp["W2"])
            s2 = scat(v2)
            relu_feat = bi > 0 and li == 0
            feat, vh = _call(
                functools.partial(_d2_body, relu_feat),
                [(NP, 32), (NP, 32)], s2, v2, dis,
                batch2d, lp["b2"][None, :], lp["gn2_w"][None, :],
                lp["gn2_b"][None, :], lp["gn2_ms"][None, :], feat)

    ipw = p["mha_ipw"]
    ipb = p["mha_ipb"]
    (out,) = _call(
        _final_body, [(NP, 32)], feat, batch2d, ipw[64:].T,
        ipb[64:][None, :], p["mha_ow"].T, p["mha_ob"][None, :])
    return out[:N]


# speedup vs baseline: 1.0118x; 1.0118x over previous
import functools

import jax
import jax.numpy as jnp
from jax import lax
from jax.experimental import pallas as pl
from jax.experimental.pallas import tpu as pltpu
from jax.experimental.pallas import tpu_sc as plsc

N_GRAPHS = 16
_F32 = jnp.float32

N_PAD = 10240
SC_CORES = 2
SC_TILES = 16
N_WORKERS = SC_CORES * SC_TILES
CHUNK = 256
NBUF = 4
PF = 2
ROWS_PER_TILE = N_PAD // SC_TILES


def _sc_mesh():
    return plsc.VectorSubcoreMesh(core_axis_name="c", subcore_axis_name="s")


def _spmm_sc(n_chunks):
    ept_rows = n_chunks

    def body(v_hbm, src_hbm, dst_hbm, zz_hbm, out_hbm, acc, sidx, didx,
             gb, gsem):
        c = lax.axis_index("c")
        s = lax.axis_index("s")
        wid = c * SC_TILES + s
        pltpu.sync_copy(zz_hbm.at[pl.ds(s * ROWS_PER_TILE, ROWS_PER_TILE)],
                        acc.at[pl.ds(s * ROWS_PER_TILE, ROWS_PER_TILE)])
        pltpu.sync_copy(src_hbm.at[pl.ds(wid * ept_rows, ept_rows)], sidx)
        pltpu.sync_copy(dst_hbm.at[pl.ds(wid * ept_rows, ept_rows)], didx)
        plsc.subcore_barrier()

        for b in range(PF):
            pltpu.make_async_copy(v_hbm.at[sidx.at[b]], gb.at[b],
                                  gsem.at[b]).start()

        def step(j, carry):
            for b in range(NBUF):
                jj = j + b
                bp = (b + PF) % NBUF
                pltpu.make_async_copy(v_hbm.at[sidx.at[jj]], gb.at[b],
                                      gsem.at[b]).wait()

                @pl.when(jj + PF < n_chunks)
                def _():
                    pltpu.make_async_copy(v_hbm.at[sidx.at[jj + PF]],
                                          gb.at[bp], gsem.at[bp]).start()

                pltpu.sync_copy(gb.at[b], acc.at[didx.at[jj]], add=True)
            return carry

        lax.fori_loop(0, n_chunks // NBUF, lambda i, cc: step(i * NBUF, cc),
                      0, unroll=False)
        plsc.subcore_barrier()
        pltpu.sync_copy(acc.at[pl.ds(s * ROWS_PER_TILE, ROWS_PER_TILE)],
                        out_hbm.at[c].at[pl.ds(s * ROWS_PER_TILE,
                                               ROWS_PER_TILE)])

    return pl.kernel(
        body,
        out_type=jax.ShapeDtypeStruct((SC_CORES, N_PAD, 32), _F32),
        mesh=_sc_mesh(),
        compiler_params=pltpu.CompilerParams(use_tc_tiling_on_sc=False),
        scratch_types=[
            pltpu.VMEM_SHARED((N_PAD, 32), _F32),
            pltpu.VMEM((ept_rows, CHUNK), jnp.int32),
            pltpu.VMEM((ept_rows, CHUNK), jnp.int32),
            pltpu.VMEM((NBUF, CHUNK, 32), _F32),
            pltpu.SemaphoreType.DMA((NBUF,)),
        ],
    )


def _deg_sc(n_chunks):
    ept_rows = n_chunks

    def body(dst_hbm, zz_hbm, out_hbm, acc, didx, ones):
        c = lax.axis_index("c")
        s = lax.axis_index("s")
        wid = c * SC_TILES + s
        pltpu.sync_copy(zz_hbm.at[pl.ds(s * ROWS_PER_TILE, ROWS_PER_TILE)],
                        acc.at[pl.ds(s * ROWS_PER_TILE, ROWS_PER_TILE)])
        pltpu.sync_copy(dst_hbm.at[pl.ds(wid * ept_rows, ept_rows)], didx)

        def initones(i, cc):
            ones[i, :] = jnp.ones((16,), _F32)
            return cc

        lax.fori_loop(0, CHUNK, initones, 0, unroll=False)
        plsc.subcore_barrier()

        def step(j, cc):
            pltpu.sync_copy(ones, acc.at[didx.at[j]], add=True)
            return cc

        lax.fori_loop(0, n_chunks, step, 0, unroll=False)
        plsc.subcore_barrier()
        pltpu.sync_copy(acc.at[pl.ds(s * ROWS_PER_TILE, ROWS_PER_TILE)],
                        out_hbm.at[c].at[pl.ds(s * ROWS_PER_TILE,
                                               ROWS_PER_TILE)])

    return pl.kernel(
        body,
        out_type=jax.ShapeDtypeStruct((SC_CORES, N_PAD, 16), _F32),
        mesh=_sc_mesh(),
        compiler_params=pltpu.CompilerParams(use_tc_tiling_on_sc=False),
        scratch_types=[
            pltpu.VMEM_SHARED((N_PAD, 16), _F32),
            pltpu.VMEM((ept_rows, CHUNK), jnp.int32),
            pltpu.VMEM((CHUNK, 16), _F32),
        ],
    )


def _batch_onehot(batch2d):
    gid = lax.broadcasted_iota(jnp.int32, (1, N_GRAPHS), 1)
    bmat = (batch2d == gid).astype(_F32)
    cnt = jnp.maximum(jnp.sum(bmat, axis=0, keepdims=True), 1.0)
    inv_cnt = (1.0 / cnt).reshape(N_GRAPHS, 1)
    return bmat, inv_cnt


def _dotg(a, b, ca, cb):
    return lax.dot_general(a, b, (((ca,), (cb,)), ((), ())),
                           preferred_element_type=_F32)


def _split(a):
    hi = a.astype(jnp.bfloat16).astype(_F32)
    return hi, a - hi


def _dot_oh(oh, b, ca, cb):
    bh, bl = _split(b)
    return _dotg(oh, bh, ca, cb) + _dotg(oh, bl, ca, cb)


def _dot3(a, b, ca, cb):
    ah, al = _split(a)
    bh, bl = _split(b)
    return (_dotg(ah, bh, ca, cb) + _dotg(ah, bl, ca, cb)
            + _dotg(al, bh, ca, cb))


def _gnorm(z, bmat, inv_cnt, w, b, ms):
    mean = _dot_oh(bmat, z, 0, 0) * inv_cnt
    o = z - _dot_oh(bmat, mean, 1, 0) * ms
    var = _dot_oh(bmat, o * o, 0, 0) * inv_cnt
    rstd = lax.rsqrt(var + 1e-5)
    return w * o * _dot_oh(bmat, rstd, 1, 0) + b


def _pre_body(deg_ref, x_ref, w0_ref, dis_ref, v0_ref):
    dis = lax.rsqrt(deg_ref[...] + 1.0)
    dis_ref[...] = dis
    v0_ref[...] = dis * _dot3(x_ref[...], w0_ref[...], 1, 0)


def _d0_body(s_ref, v_ref, dis_ref, batch_ref, b0_ref, gw_ref, gb_ref,
             gms_ref, h_ref, vh_ref):
    bmat, inv_cnt = _batch_onehot(batch_ref[...])
    a = dis_ref[...] * (s_ref[...] + v_ref[...]) + b0_ref[...]
    h = jax.nn.relu(_gnorm(a, bmat, inv_cnt, gw_ref[...], gb_ref[...],
                           gms_ref[...]))
    h_ref[...] = h
    vh_ref[...] = dis_ref[...] * h


def _d1_body(s_ref, vh_ref, dis_ref, batch_ref, w1_ref, b1_ref, gw_ref,
             gb_ref, gms_ref, w2_ref, v2_ref):
    bmat, inv_cnt = _batch_onehot(batch_ref[...])
    a = dis_ref[...] * (s_ref[...] + vh_ref[...])
    z = _dot3(a, w1_ref[...], 1, 0) + b1_ref[...]
    t = jax.nn.relu(_gnorm(z, bmat, inv_cnt, gw_ref[...], gb_ref[...],
                           gms_ref[...]))
    v2_ref[...] = dis_ref[...] * _dot3(t, w2_ref[...], 1, 0)


def _d2_body(relu_feat, s_ref, v2_ref, dis_ref, batch_ref, b2_ref, gw_ref,
             gb_ref, gms_ref, feat_ref, feato_ref, vh_ref):
    bmat, inv_cnt = _batch_onehot(batch_ref[...])
    a = dis_ref[...] * (s_ref[...] + v2_ref[...]) + b2_ref[...]
    t2 = jax.nn.relu(_gnorm(a, bmat, inv_cnt, gw_ref[...], gb_ref[...],
                            gms_ref[...]))
    fin = jax.nn.relu(feat_ref[...]) if relu_feat else feat_ref[...]
    f = fin + t2
    feato_ref[...] = f
    vh_ref[...] = dis_ref[...] * jax.nn.relu(f)


def _final_body(feat_ref, batch_ref, wvt_ref, bv_ref, owt_ref, ob_ref,
                out_ref):
    h = jax.nn.relu(feat_ref[...])
    batch2d = batch_ref[...]
    bmat, _ = _batch_onehot(batch2d)
    rows = []
    for g in range(N_GRAPHS):
        hg = jnp.where(batch2d == g, h, -1.0)
        rows.append(jnp.max(hg, axis=0, keepdims=True))
    pooled = jnp.concatenate(rows, axis=0)
    go = _dot3(_dot3(pooled, wvt_ref[...], 1, 0) + bv_ref[...],
               owt_ref[...], 1, 0) + ob_ref[...]
    out_ref[...] = _dot_oh(bmat, go, 1, 0)


def _call(body, out_shapes, *args):
    return pl.pallas_call(
        body,
        out_shape=[jax.ShapeDtypeStruct(s, _F32) for s in out_shapes],
    )(*args)


def kernel(x, edge_index, batch, params):
    N = x.shape[0]
    E = edge_index.shape[1]
    NP = N_PAD

    xp = jnp.zeros((NP, x.shape[1]), _F32).at[:N].set(x)
    batch2d = jnp.concatenate(
        [batch, jnp.full((NP - N,), N_GRAPHS, batch.dtype)])[:, None]
    n_chunks = -(-E // (CHUNK * N_WORKERS))
    n_chunks = -(-n_chunks // NBUF) * NBUF
    e_pad = n_chunks * CHUNK * N_WORKERS
    idx_pad = jnp.full((2, e_pad - E), N, edge_index.dtype)
    srcp = jnp.concatenate([edge_index, idx_pad], axis=1)[0].reshape(
        N_WORKERS * n_chunks, CHUNK)
    dstp = jnp.concatenate([edge_index, idx_pad], axis=1)[1].reshape(
        N_WORKERS * n_chunks, CHUNK)
    zz32 = jnp.zeros((NP, 32), _F32)
    zz16 = jnp.zeros((NP, 16), _F32)

    spmm = _spmm_sc(n_chunks)
    degk = _deg_sc(n_chunks)

    def scat(v):
        sp = spmm(v, srcp, dstp, zz32)
        return sp[0] + sp[1]

    deg_part = degk(dstp, zz16)
    degsum = (deg_part[0, :, 0] + deg_part[1, :, 0])[:, None]

    p = params
    dis, v0 = _call(_pre_body, [(NP, 1), (NP, 32)], degsum, xp,
                    p["conv0_W"])

    s0 = scat(v0)
    h, vh = _call(
        _d0_body, [(NP, 32), (NP, 32)], s0, v0, dis, batch2d,
        p["conv0_b"][None, :], p["gn0_w"][None, :], p["gn0_b"][None, :],
        p["gn0_ms"][None, :])

    feat = h
    for bi, block in enumerate(p["blocks"]):
        for li, lp in enumerate(block):
            s1 = scat(vh)
            (v2,) = _call(
                _d1_body, [(NP, 32)], s1, vh, dis, batch2d, lp["W1"],
                lp["b1"][None, :], lp["gn1_w"][None, :],
                lp["gn1_b"][None, :], lp["gn1_ms"][None, :], lp["W2"])
            s2 = scat(v2)
            relu_feat = bi > 0 and li == 0
            feat, vh = _call(
                functools.partial(_d2_body, relu_feat),
                [(NP, 32), (NP, 32)], s2, v2, dis,
                batch2d, lp["b2"][None, :], lp["gn2_w"][None, :],
                lp["gn2_b"][None, :], lp["gn2_ms"][None, :], feat)

    ipw = p["mha_ipw"]
    ipb = p["mha_ipb"]
    (out,) = _call(
        _final_body, [(NP, 32)], feat, batch2d, ipw[64:].T,
        ipb[64:][None, :], p["mha_ow"].T, p["mha_ob"][None, :])
    return out[:N]


# speedup vs baseline: 1.0356x; 1.0235x over previous
import functools

import jax
import jax.numpy as jnp
from jax import lax
from jax.experimental import pallas as pl
from jax.experimental.pallas import tpu as pltpu
from jax.experimental.pallas import tpu_sc as plsc

N_GRAPHS = 16
_F32 = jnp.float32

N_PAD = 10240
SC_CORES = 2
SC_TILES = 16
N_WORKERS = SC_CORES * SC_TILES
CHUNK = 256
NBUF = 4
PF = 2
ROWS_PER_TILE = N_PAD // SC_TILES


def _sc_mesh():
    return plsc.VectorSubcoreMesh(core_axis_name="c", subcore_axis_name="s")


def _spmm_sc(n_chunks):
    ept_rows = n_chunks

    def body(v_hbm, src_hbm, dst_hbm, zz_hbm, out_hbm, acc, sidx, didx,
             gb, gsem):
        c = lax.axis_index("c")
        s = lax.axis_index("s")
        wid = c * SC_TILES + s
        z_cp = pltpu.make_async_copy(
            zz_hbm.at[pl.ds(s * ROWS_PER_TILE, ROWS_PER_TILE)],
            acc.at[pl.ds(s * ROWS_PER_TILE, ROWS_PER_TILE)], gsem.at[0])
        z_cp.start()
        s_cp = pltpu.make_async_copy(
            src_hbm.at[pl.ds(wid * ept_rows, ept_rows)], sidx, gsem.at[1])
        s_cp.start()
        d_cp = pltpu.make_async_copy(
            dst_hbm.at[pl.ds(wid * ept_rows, ept_rows)], didx, gsem.at[2])
        d_cp.start()
        z_cp.wait()
        s_cp.wait()
        d_cp.wait()
        plsc.subcore_barrier()

        for b in range(PF):
            pltpu.make_async_copy(v_hbm.at[sidx.at[b]], gb.at[b],
                                  gsem.at[b]).start()

        def step(j, carry):
            for b in range(NBUF):
                jj = j + b
                bp = (b + PF) % NBUF
                pltpu.make_async_copy(v_hbm.at[sidx.at[jj]], gb.at[b],
                                      gsem.at[b]).wait()

                @pl.when(jj + PF < n_chunks)
                def _():
                    pltpu.make_async_copy(v_hbm.at[sidx.at[jj + PF]],
                                          gb.at[bp], gsem.at[bp]).start()

                pltpu.sync_copy(gb.at[b], acc.at[didx.at[jj]], add=True)
            return carry

        lax.fori_loop(0, n_chunks // NBUF, lambda i, cc: step(i * NBUF, cc),
                      0, unroll=False)
        plsc.subcore_barrier()
        pltpu.sync_copy(acc.at[pl.ds(s * ROWS_PER_TILE, ROWS_PER_TILE)],
                        out_hbm.at[c].at[pl.ds(s * ROWS_PER_TILE,
                                               ROWS_PER_TILE)])

    return pl.kernel(
        body,
        out_type=jax.ShapeDtypeStruct((SC_CORES, N_PAD, 32), _F32),
        mesh=_sc_mesh(),
        compiler_params=pltpu.CompilerParams(use_tc_tiling_on_sc=False),
        scratch_types=[
            pltpu.VMEM_SHARED((N_PAD, 32), _F32),
            pltpu.VMEM((ept_rows, CHUNK), jnp.int32),
            pltpu.VMEM((ept_rows, CHUNK), jnp.int32),
            pltpu.VMEM((NBUF, CHUNK, 32), _F32),
            pltpu.SemaphoreType.DMA((NBUF,)),
        ],
    )


def _deg_sc(n_chunks):
    ept_rows = n_chunks

    def body(dst_hbm, zz_hbm, out_hbm, acc, didx, ones):
        c = lax.axis_index("c")
        s = lax.axis_index("s")
        wid = c * SC_TILES + s
        pltpu.sync_copy(zz_hbm.at[pl.ds(s * ROWS_PER_TILE, ROWS_PER_TILE)],
                        acc.at[pl.ds(s * ROWS_PER_TILE, ROWS_PER_TILE)])
        pltpu.sync_copy(dst_hbm.at[pl.ds(wid * ept_rows, ept_rows)], didx)

        def initones(i, cc):
            ones[i, :] = jnp.ones((16,), _F32)
            return cc

        lax.fori_loop(0, CHUNK, initones, 0, unroll=False)
        plsc.subcore_barrier()

        def step(j, cc):
            pltpu.sync_copy(ones, acc.at[didx.at[j]], add=True)
            return cc

        lax.fori_loop(0, n_chunks, step, 0, unroll=False)
        plsc.subcore_barrier()
        pltpu.sync_copy(acc.at[pl.ds(s * ROWS_PER_TILE, ROWS_PER_TILE)],
                        out_hbm.at[c].at[pl.ds(s * ROWS_PER_TILE,
                                               ROWS_PER_TILE)])

    return pl.kernel(
        body,
        out_type=jax.ShapeDtypeStruct((SC_CORES, N_PAD, 16), _F32),
        mesh=_sc_mesh(),
        compiler_params=pltpu.CompilerParams(use_tc_tiling_on_sc=False),
        scratch_types=[
            pltpu.VMEM_SHARED((N_PAD, 16), _F32),
            pltpu.VMEM((ept_rows, CHUNK), jnp.int32),
            pltpu.VMEM((CHUNK, 16), _F32),
        ],
    )


def _batch_onehot(batch2d):
    gid = lax.broadcasted_iota(jnp.int32, (1, N_GRAPHS), 1)
    bmat = (batch2d == gid).astype(_F32)
    cnt = jnp.maximum(jnp.sum(bmat, axis=0, keepdims=True), 1.0)
    inv_cnt = (1.0 / cnt).reshape(N_GRAPHS, 1)
    return bmat, inv_cnt


def _dotg(a, b, ca, cb):
    return lax.dot_general(a, b, (((ca,), (cb,)), ((), ())),
                           preferred_element_type=_F32)


def _split(a):
    hi = a.astype(jnp.bfloat16).astype(_F32)
    return hi, a - hi


def _dot_oh(oh, b, ca, cb):
    bh, bl = _split(b)
    return _dotg(oh, bh, ca, cb) + _dotg(oh, bl, ca, cb)


def _dot3(a, b, ca, cb):
    ah, al = _split(a)
    bh, bl = _split(b)
    return (_dotg(ah, bh, ca, cb) + _dotg(ah, bl, ca, cb)
            + _dotg(al, bh, ca, cb))


def _gnorm(z, bmat, inv_cnt, w, b, ms):
    mean = _dot_oh(bmat, z, 0, 0) * inv_cnt
    o = z - _dot_oh(bmat, mean, 1, 0) * ms
    var = _dot_oh(bmat, o * o, 0, 0) * inv_cnt
    rstd = lax.rsqrt(var + 1e-5)
    return w * o * _dot_oh(bmat, rstd, 1, 0) + b


def _pre_body(deg_ref, x_ref, w0_ref, dis_ref, v0_ref):
    dis = lax.rsqrt(deg_ref[...] + 1.0)
    dis_ref[...] = dis
    v0_ref[...] = dis * _dot3(x_ref[...], w0_ref[...], 1, 0)


def _d0_body(s_ref, v_ref, dis_ref, batch_ref, b0_ref, gw_ref, gb_ref,
             gms_ref, h_ref, vh_ref):
    bmat, inv_cnt = _batch_onehot(batch_ref[...])
    a = dis_ref[...] * (s_ref[0] + s_ref[1] + v_ref[...]) + b0_ref[...]
    h = jax.nn.relu(_gnorm(a, bmat, inv_cnt, gw_ref[...], gb_ref[...],
                           gms_ref[...]))
    h_ref[...] = h
    vh_ref[...] = dis_ref[...] * h


def _d1_body(s_ref, vh_ref, dis_ref, batch_ref, w1_ref, b1_ref, gw_ref,
             gb_ref, gms_ref, w2_ref, v2_ref):
    bmat, inv_cnt = _batch_onehot(batch_ref[...])
    a = dis_ref[...] * (s_ref[0] + s_ref[1] + vh_ref[...])
    z = _dot3(a, w1_ref[...], 1, 0) + b1_ref[...]
    t = jax.nn.relu(_gnorm(z, bmat, inv_cnt, gw_ref[...], gb_ref[...],
                           gms_ref[...]))
    v2_ref[...] = dis_ref[...] * _dot3(t, w2_ref[...], 1, 0)


def _d2_body(relu_feat, s_ref, v2_ref, dis_ref, batch_ref, b2_ref, gw_ref,
             gb_ref, gms_ref, feat_ref, feato_ref, vh_ref):
    bmat, inv_cnt = _batch_onehot(batch_ref[...])
    a = dis_ref[...] * (s_ref[...] + v2_ref[...]) + b2_ref[...]
    t2 = jax.nn.relu(_gnorm(a, bmat, inv_cnt, gw_ref[...], gb_ref[...],
                            gms_ref[...]))
    fin = jax.nn.relu(feat_ref[...]) if relu_feat else feat_ref[...]
    f = fin + t2
    feato_ref[...] = f
    vh_ref[...] = dis_ref[...] * jax.nn.relu(f)


def _final_body(feat_ref, batch_ref, wvt_ref, bv_ref, owt_ref, ob_ref,
                out_ref):
    h = jax.nn.relu(feat_ref[...])
    batch2d = batch_ref[...]
    bmat, _ = _batch_onehot(batch2d)
    rows = []
    for g in range(N_GRAPHS):
        hg = jnp.where(batch2d == g, h, -1.0)
        rows.append(jnp.max(hg, axis=0, keepdims=True))
    pooled = jnp.concatenate(rows, axis=0)
    go = _dot3(_dot3(pooled, wvt_ref[...], 1, 0) + bv_ref[...],
               owt_ref[...], 1, 0) + ob_ref[...]
    out_ref[...] = _dot_oh(bmat, go, 1, 0)


def _call(body, out_shapes, *args):
    return pl.pallas_call(
        body,
        out_shape=[jax.ShapeDtypeStruct(s, _F32) for s in out_shapes],
    )(*args)


def kernel(x, edge_index, batch, params):
    N = x.shape[0]
    E = edge_index.shape[1]
    NP = N_PAD

    xp = jnp.zeros((NP, x.shape[1]), _F32).at[:N].set(x)
    batch2d = jnp.concatenate(
        [batch, jnp.full((NP - N,), N_GRAPHS, batch.dtype)])[:, None]
    n_chunks = -(-E // (CHUNK * N_WORKERS))
    n_chunks = -(-n_chunks // NBUF) * NBUF
    e_pad = n_chunks * CHUNK * N_WORKERS
    idx_pad = jnp.full((2, e_pad - E), N, edge_index.dtype)
    srcp = jnp.concatenate([edge_index, idx_pad], axis=1)[0].reshape(
        N_WORKERS * n_chunks, CHUNK)
    dstp = jnp.concatenate([edge_index, idx_pad], axis=1)[1].reshape(
        N_WORKERS * n_chunks, CHUNK)
    zz32 = jnp.zeros((NP, 32), _F32)
    zz16 = jnp.zeros((NP, 16), _F32)

    spmm = _spmm_sc(n_chunks)
    degk = _deg_sc(n_chunks)

    def scat(v):
        return spmm(v, srcp, dstp, zz32)

    deg_part = degk(dstp, zz16)
    degsum = (deg_part[0, :, 0] + deg_part[1, :, 0])[:, None]

    p = params
    dis, v0 = _call(_pre_body, [(NP, 1), (NP, 32)], degsum, xp,
                    p["conv0_W"])

    s0 = scat(v0)
    h, vh = _call(
        _d0_body, [(NP, 32), (NP, 32)], s0, v0, dis, batch2d,
        p["conv0_b"][None, :], p["gn0_w"][None, :], p["gn0_b"][None, :],
        p["gn0_ms"][None, :])

    feat = h
    for bi, block in enumerate(p["blocks"]):
        for li, lp in enumerate(block):
            s1 = scat(vh)
            (v2,) = _call(
                _d1_body, [(NP, 32)], s1, vh, dis, batch2d, lp["W1"],
                lp["b1"][None, :], lp["gn1_w"][None, :],
                lp["gn1_b"][None, :], lp["gn1_ms"][None, :], lp["W2"])
            s2p = scat(v2)
            s2 = s2p[0] + s2p[1]
            relu_feat = bi > 0 and li == 0
            feat, vh = _call(
                functools.partial(_d2_body, relu_feat),
                [(NP, 32), (NP, 32)], s2, v2, dis,
                batch2d, lp["b2"][None, :], lp["gn2_w"][None, :],
                lp["gn2_b"][None, :], lp["gn2_ms"][None, :], feat)

    ipw = p["mha_ipw"]
    ipb = p["mha_ipb"]
    (out,) = _call(
        _final_body, [(NP, 32)], feat, batch2d, ipw[64:].T,
        ipb[64:][None, :], p["mha_ow"].T, p["mha_ob"][None, :])
    return out[:N]
